# baseline (speedup 1.0000x reference)
import functools

import jax
import jax.numpy as jnp
from jax import lax
from jax.experimental import pallas as pl
from jax.experimental.pallas import tpu as pltpu
from jax.experimental.pallas import tpu_sc as plsc

NF = 26
B = 4096
V = 100000
D = 64
NW = 32
CHUNK = B // NW
BB = 256
GRID = B // BB


def _sc_gather(tab, idx):
    mesh = plsc.VectorSubcoreMesh(core_axis_name="c", subcore_axis_name="s")

    @functools.partial(
        pl.kernel,
        mesh=mesh,
        out_type=jax.ShapeDtypeStruct((NF, B, 2 * D), jnp.float32),
        scratch_types=[
            pltpu.VMEM((NF, CHUNK), jnp.int32),
            pltpu.VMEM((CHUNK, 2 * D), jnp.float32),
            pltpu.VMEM((CHUNK, 2 * D), jnp.float32),
            pltpu.SemaphoreType.DMA,
            pltpu.SemaphoreType.DMA,
        ],
    )
    def k(tab_hbm, idx_hbm, out_hbm, idx_v, rows0, rows1, sem0, sem1):
        wid = lax.axis_index("s") * 2 + lax.axis_index("c")
        base = wid * CHUNK
        pltpu.sync_copy(idx_hbm.at[:, pl.ds(base, CHUNK)], idx_v)

        def shift(t, _):
            for j in range(CHUNK // 16):
                sl = pl.ds(j * 16, 16)
                idx_v[t, sl] = lax.shift_right_logical(idx_v[t, sl], 1)
            return _
        lax.fori_loop(0, NF, shift, 0)

        def step(t, _):
            u0 = 2 * t
            u1 = 2 * t + 1
            g0 = pltpu.async_copy(tab_hbm.at[u0].at[idx_v.at[u0]], rows0, sem0)
            g1 = pltpu.async_copy(tab_hbm.at[u1].at[idx_v.at[u1]], rows1, sem1)
            g0.wait()
            pltpu.sync_copy(rows0, out_hbm.at[u0, pl.ds(base, CHUNK)])
            g1.wait()
            pltpu.sync_copy(rows1, out_hbm.at[u1, pl.ds(base, CHUNK)])
            return _
        lax.fori_loop(0, NF // 2, step, 0)

    return k(tab, idx)


def _dense_body(dx_ref, ly_ref, idx_ref, bw0, bb0, bw1, bb1, bw2, bb2,
                w0a, *rest):
    wbs = rest[:NF]
    tb0, tw1, tb1, tw2, tb2, out_ref = rest[NF:]

    f32 = jnp.float32
    x = dx_ref[...]
    x = jnp.maximum(jnp.dot(x, bw0[...], preferred_element_type=f32) + bb0[...], 0.0)
    x = jnp.maximum(jnp.dot(x, bw1[...], preferred_element_type=f32) + bb1[...], 0.0)
    x = jnp.maximum(jnp.dot(x, bw2[...], preferred_element_type=f32) + bb2[...], 0.0)

    odd = (idx_ref[...] & 1)[:, :, None] == 1
    ly2 = ly_ref[...]
    ly = jnp.where(odd, ly2[:, :, D:], ly2[:, :, :D])
    t3 = jnp.concatenate([x[None], ly], axis=0)

    acc = lax.dot_general(w0a[...], x, (((1,), (1,)), ((), ())),
                          preferred_element_type=f32)
    acc = acc + tb0[...]
    for i in range(1, 27):
        p = jnp.sum(t3[i] * t3[:i], axis=-1)
        acc = acc + lax.dot_general(wbs[i - 1][...], p,
                                    (((1,), (0,)), ((), ())),
                                    preferred_element_type=f32)
    h = jnp.maximum(acc, 0.0)
    h = jnp.maximum(jnp.dot(tw1[...], h, preferred_element_type=f32) + tb1[...], 0.0)
    h = jnp.dot(tw2[...], h, preferred_element_type=f32) + tb2[...]
    out_ref[...] = 1.0 / (1.0 + jnp.exp(-h))


def _dense(dense_x, ly, idx, bw0t, bb0, bw1t, bb1, bw2t, bb2,
           w0a, wbs, tb0, tw1, tb1, tw2, tb2):
    full = lambda s: pl.BlockSpec(s, lambda i: (0,) * len(s))
    in_specs = [
        pl.BlockSpec((BB, 13), lambda i: (i, 0)),
        pl.BlockSpec((NF, BB, 2 * D), lambda i: (0, i, 0)),
        pl.BlockSpec((NF, BB), lambda i: (0, i)),
        full((13, 512)), full((1, 512)),
        full((512, 256)), full((1, 256)),
        full((256, 64)), full((1, 64)),
        full((512, 64)),
    ]
    in_specs += [full((512, i)) for i in range(1, 27)]
    in_specs += [full((512, 1)), full((256, 512)), full((256, 1)),
                 full((1, 256)), full((1, 1))]
    out = pl.pallas_call(
        _dense_body,
        grid=(GRID,),
        in_specs=in_specs,
        out_specs=pl.BlockSpec((1, BB), lambda i: (0, i)),
        out_shape=jax.ShapeDtypeStruct((1, B), jnp.float32),
        compiler_params=pltpu.CompilerParams(
            dimension_semantics=("arbitrary",)),
    )(dense_x, ly, idx, bw0t, bb0, bw1t, bb1, bw2t, bb2, w0a, *wbs,
      tb0, tw1, tb1, tw2, tb2)
    return out


def kernel(dense_x, lS_i, emb_tables,
           bot_w0, bot_b0, bot_w1, bot_b1, bot_w2, bot_b2,
           top_w0, top_b0, top_w1, top_b1, top_w2, top_b2):
    idx = lS_i.astype(jnp.int32)
    t1 = jnp.transpose(emb_tables, (0, 2, 1))
    t2 = t1.reshape(NF, D, V // 2, 2)
    tabp = jnp.transpose(t2, (0, 2, 3, 1)).reshape(NF, V // 2, 2 * D)
    ly = _sc_gather(tabp, idx)

    bw0t, bw1t, bw2t = bot_w0.T, bot_w1.T, bot_w2.T
    bb0, bb1, bb2 = (bot_b0.reshape(1, -1), bot_b1.reshape(1, -1),
                     bot_b2.reshape(1, -1))
    w0a = top_w0[:, :D]
    offs = [i * (i - 1) // 2 for i in range(27)]
    wbs = [top_w0[:, D + offs[i]: D + offs[i] + i] for i in range(1, 27)]
    tb0 = top_b0.reshape(-1, 1)
    tb1 = top_b1.reshape(-1, 1)
    tb2 = top_b2.reshape(-1, 1)

    out = _dense(dense_x, ly, idx, bw0t, bb0, bw1t, bb1, bw2t, bb2,
                 w0a, wbs, tb0, top_w1, tb1, top_w2, tb2)
    return out.reshape(B, 1)


# speedup vs baseline: 1.2918x; 1.2918x over previous
import functools

import jax
import jax.numpy as jnp
from jax import lax
from jax.experimental import pallas as pl
from jax.experimental.pallas import tpu as pltpu
from jax.experimental.pallas import tpu_sc as plsc

NF = 26
B = 4096
V = 100000
D = 64
NW = 32
CHUNK = B // NW
BB = 256
GRID = B // BB


def _sc_gather(tab, idx):
    mesh = plsc.VectorSubcoreMesh(core_axis_name="c", subcore_axis_name="s")

    @functools.partial(
        pl.kernel,
        mesh=mesh,
        out_type=jax.ShapeDtypeStruct((NF, B, 2 * D), jnp.float32),
        scratch_types=[
            pltpu.VMEM((NF, CHUNK), jnp.int32),
            pltpu.VMEM((CHUNK, D), jnp.float32),
            pltpu.VMEM((CHUNK, D), jnp.float32),
            pltpu.SemaphoreType.DMA,
            pltpu.SemaphoreType.DMA,
        ],
        compiler_params=pltpu.CompilerParams(use_tc_tiling_on_sc=False),
    )
    def k(tab_hbm, idx_hbm, out_hbm, idx_v, rows0, rows1, sem0, sem1):
        wid = lax.axis_index("s") * 2 + lax.axis_index("c")
        base = wid * CHUNK
        pltpu.sync_copy(idx_hbm.at[:, pl.ds(base, CHUNK)], idx_v)

        def step(t, _):
            u0 = 2 * t
            u1 = 2 * t + 1
            g0 = pltpu.async_copy(tab_hbm.at[u0].at[idx_v.at[u0]], rows0, sem0)
            g1 = pltpu.async_copy(tab_hbm.at[u1].at[idx_v.at[u1]], rows1, sem1)
            g0.wait()
            pltpu.sync_copy(rows0, out_hbm.at[u0, pl.ds(base, CHUNK), pl.ds(0, D)])
            g1.wait()
            pltpu.sync_copy(rows1, out_hbm.at[u1, pl.ds(base, CHUNK), pl.ds(0, D)])
            return _
        lax.fori_loop(0, NF // 2, step, 0)

    return k(tab, idx)


def _dense_body(dx_ref, ly_ref, idx_ref, bw0, bb0, bw1, bb1, bw2, bb2,
                w0a, *rest):
    wbs = rest[:NF]
    tb0, tw1, tb1, tw2, tb2, out_ref = rest[NF:]

    f32 = jnp.float32
    x = dx_ref[...]
    x = jnp.maximum(jnp.dot(x, bw0[...], preferred_element_type=f32) + bb0[...], 0.0)
    x = jnp.maximum(jnp.dot(x, bw1[...], preferred_element_type=f32) + bb1[...], 0.0)
    x = jnp.maximum(jnp.dot(x, bw2[...], preferred_element_type=f32) + bb2[...], 0.0)

    ly = ly_ref[:, :, 0:D]
    t3 = jnp.concatenate([x[None], ly], axis=0)

    acc = lax.dot_general(w0a[...], x, (((1,), (1,)), ((), ())),
                          preferred_element_type=f32)
    acc = acc + tb0[...]
    for i in range(1, 27):
        p = jnp.sum(t3[i] * t3[:i], axis=-1)
        acc = acc + lax.dot_general(wbs[i - 1][...], p,
                                    (((1,), (0,)), ((), ())),
                                    preferred_element_type=f32)
    h = jnp.maximum(acc, 0.0)
    h = jnp.maximum(jnp.dot(tw1[...], h, preferred_element_type=f32) + tb1[...], 0.0)
    h = jnp.dot(tw2[...], h, preferred_element_type=f32) + tb2[...]
    out_ref[...] = 1.0 / (1.0 + jnp.exp(-h))


def _dense(dense_x, ly, idx, bw0t, bb0, bw1t, bb1, bw2t, bb2,
           w0a, wbs, tb0, tw1, tb1, tw2, tb2):
    full = lambda s: pl.BlockSpec(s, lambda i: (0,) * len(s))
    in_specs = [
        pl.BlockSpec((BB, 13), lambda i: (i, 0)),
        pl.BlockSpec((NF, BB, 2 * D), lambda i: (0, i, 0)),
        pl.BlockSpec((NF, BB), lambda i: (0, i)),
        full((13, 512)), full((1, 512)),
        full((512, 256)), full((1, 256)),
        full((256, 64)), full((1, 64)),
        full((512, 64)),
    ]
    in_specs += [full((512, i)) for i in range(1, 27)]
    in_specs += [full((512, 1)), full((256, 512)), full((256, 1)),
                 full((1, 256)), full((1, 1))]
    out = pl.pallas_call(
        _dense_body,
        grid=(GRID,),
        in_specs=in_specs,
        out_specs=pl.BlockSpec((1, BB), lambda i: (0, i)),
        out_shape=jax.ShapeDtypeStruct((1, B), jnp.float32),
        compiler_params=pltpu.CompilerParams(
            dimension_semantics=("arbitrary",)),
    )(dense_x, ly, idx, bw0t, bb0, bw1t, bb1, bw2t, bb2, w0a, *wbs,
      tb0, tw1, tb1, tw2, tb2)
    return out


def kernel(dense_x, lS_i, emb_tables,
           bot_w0, bot_b0, bot_w1, bot_b1, bot_w2, bot_b2,
           top_w0, top_b0, top_w1, top_b1, top_w2, top_b2):
    idx = lS_i.astype(jnp.int32)
    ly = _sc_gather(emb_tables, idx)

    bw0t, bw1t, bw2t = bot_w0.T, bot_w1.T, bot_w2.T
    bb0, bb1, bb2 = (bot_b0.reshape(1, -1), bot_b1.reshape(1, -1),
                     bot_b2.reshape(1, -1))
    w0a = top_w0[:, :D]
    offs = [i * (i - 1) // 2 for i in range(27)]
    wbs = [top_w0[:, D + offs[i]: D + offs[i] + i] for i in range(1, 27)]
    tb0 = top_b0.reshape(-1, 1)
    tb1 = top_b1.reshape(-1, 1)
    tb2 = top_b2.reshape(-1, 1)

    out = _dense(dense_x, ly, idx, bw0t, bb0, bw1t, bb1, bw2t, bb2,
                 w0a, wbs, tb0, top_w1, tb1, top_w2, tb2)
    return out.reshape(B, 1)


# speedup vs baseline: 1.2930x; 1.0009x over previous
import functools

import jax
import jax.numpy as jnp
from jax import lax
from jax.experimental import pallas as pl
from jax.experimental.pallas import tpu as pltpu
from jax.experimental.pallas import tpu_sc as plsc

NF = 26
B = 4096
V = 100000
D = 64
NW = 32
CHUNK = B // NW
BB = 256
GRID = B // BB


def _sc_gather(tab, idx):
    mesh = plsc.VectorSubcoreMesh(core_axis_name="c", subcore_axis_name="s")

    @functools.partial(
        pl.kernel,
        mesh=mesh,
        out_type=jax.ShapeDtypeStruct((NF, B, 2 * D), jnp.float32),
        scratch_types=[
            pltpu.VMEM((NF, CHUNK), jnp.int32),
            pltpu.VMEM((CHUNK, D), jnp.float32),
            pltpu.VMEM((CHUNK, D), jnp.float32),
            pltpu.SemaphoreType.DMA,
            pltpu.SemaphoreType.DMA,
        ],
        compiler_params=pltpu.CompilerParams(use_tc_tiling_on_sc=False),
    )
    def k(tab_hbm, idx_hbm, out_hbm, idx_v, rows0, rows1, sem0, sem1):
        wid = lax.axis_index("s") * 2 + lax.axis_index("c")
        base = wid * CHUNK
        pltpu.sync_copy(idx_hbm.at[:, pl.ds(base, CHUNK)], idx_v)

        def step(t, _):
            u0 = 2 * t
            u1 = 2 * t + 1
            g0 = pltpu.async_copy(tab_hbm.at[u0].at[idx_v.at[u0]], rows0, sem0)
            g1 = pltpu.async_copy(tab_hbm.at[u1].at[idx_v.at[u1]], rows1, sem1)
            g0.wait()
            pltpu.sync_copy(rows0, out_hbm.at[u0, pl.ds(base, CHUNK), pl.ds(0, D)])
            g1.wait()
            pltpu.sync_copy(rows1, out_hbm.at[u1, pl.ds(base, CHUNK), pl.ds(0, D)])
            return _
        lax.fori_loop(0, NF // 2, step, 0)

    return k(tab, idx)


def _dense_body(dx_ref, ly_ref, bw0, bb0, bw1, bb1, bw2, bb2,
                w0a, *rest):
    wbs = rest[:NF]
    tb0, tw1, tb1, tw2, tb2, out_ref = rest[NF:]

    f32 = jnp.float32
    x = dx_ref[...]
    x = jnp.maximum(jnp.dot(x, bw0[...], preferred_element_type=f32) + bb0[...], 0.0)
    x = jnp.maximum(jnp.dot(x, bw1[...], preferred_element_type=f32) + bb1[...], 0.0)
    x = jnp.maximum(jnp.dot(x, bw2[...], preferred_element_type=f32) + bb2[...], 0.0)

    ly = ly_ref[:, :, 0:D]
    t3 = jnp.concatenate([x[None], ly], axis=0)

    acc = lax.dot_general(w0a[...], x, (((1,), (1,)), ((), ())),
                          preferred_element_type=f32)
    acc = acc + tb0[...]
    for i in range(1, 27):
        p = jnp.sum(t3[i] * t3[:i], axis=-1)
        acc = acc + lax.dot_general(wbs[i - 1][...], p,
                                    (((1,), (0,)), ((), ())),
                                    preferred_element_type=f32)
    h = jnp.maximum(acc, 0.0)
    h = jnp.maximum(jnp.dot(tw1[...], h, preferred_element_type=f32) + tb1[...], 0.0)
    h = jnp.dot(tw2[...], h, preferred_element_type=f32) + tb2[...]
    out_ref[...] = 1.0 / (1.0 + jnp.exp(-h))


def _dense(dense_x, ly, bw0t, bb0, bw1t, bb1, bw2t, bb2,
           w0a, wbs, tb0, tw1, tb1, tw2, tb2):
    full = lambda s: pl.BlockSpec(s, lambda i: (0,) * len(s))
    in_specs = [
        pl.BlockSpec((BB, 13), lambda i: (i, 0)),
        pl.BlockSpec((NF, BB, 2 * D), lambda i: (0, i, 0)),
        full((13, 512)), full((1, 512)),
        full((512, 256)), full((1, 256)),
        full((256, 64)), full((1, 64)),
        full((512, 64)),
    ]
    in_specs += [full((512, i)) for i in range(1, 27)]
    in_specs += [full((512, 1)), full((256, 512)), full((256, 1)),
                 full((1, 256)), full((1, 1))]
    out = pl.pallas_call(
        _dense_body,
        grid=(GRID,),
        in_specs=in_specs,
        out_specs=pl.BlockSpec((1, BB), lambda i: (0, i)),
        out_shape=jax.ShapeDtypeStruct((1, B), jnp.float32),
        compiler_params=pltpu.CompilerParams(
            dimension_semantics=("arbitrary",)),
    )(dense_x, ly, bw0t, bb0, bw1t, bb1, bw2t, bb2, w0a, *wbs,
      tb0, tw1, tb1, tw2, tb2)
    return out


def kernel(dense_x, lS_i, emb_tables,
           bot_w0, bot_b0, bot_w1, bot_b1, bot_w2, bot_b2,
           top_w0, top_b0, top_w1, top_b1, top_w2, top_b2):
    idx = lS_i.astype(jnp.int32)
    ly = _sc_gather(emb_tables, idx)

    bw0t, bw1t, bw2t = bot_w0.T, bot_w1.T, bot_w2.T
    bb0, bb1, bb2 = (bot_b0.reshape(1, -1), bot_b1.reshape(1, -1),
                     bot_b2.reshape(1, -1))
    w0a = top_w0[:, :D]
    offs = [i * (i - 1) // 2 for i in range(27)]
    wbs = [top_w0[:, D + offs[i]: D + offs[i] + i] for i in range(1, 27)]
    tb0 = top_b0.reshape(-1, 1)
    tb1 = top_b1.reshape(-1, 1)
    tb2 = top_b2.reshape(-1, 1)

    out = _dense(dense_x, ly, bw0t, bb0, bw1t, bb1, bw2t, bb2,
                 w0a, wbs, tb0, top_w1, tb1, top_w2, tb2)
    return out.reshape(B, 1)


# speedup vs baseline: 1.2979x; 1.0038x over previous
import functools

import jax
import jax.numpy as jnp
from jax import lax
from jax.experimental import pallas as pl
from jax.experimental.pallas import tpu as pltpu
from jax.experimental.pallas import tpu_sc as plsc

NF = 26
B = 4096
V = 100000
D = 64
NW = 32
CHUNK = B // NW
BB = 256
GRID = B // BB


def _sc_gather(tab, idx):
    mesh = plsc.VectorSubcoreMesh(core_axis_name="c", subcore_axis_name="s")

    @functools.partial(
        pl.kernel,
        mesh=mesh,
        out_type=jax.ShapeDtypeStruct((NF, B, 2 * D), jnp.float32),
        scratch_types=[
            pltpu.VMEM((NF, CHUNK), jnp.int32),
            pltpu.VMEM((CHUNK, D), jnp.float32),
            pltpu.VMEM((CHUNK, D), jnp.float32),
            pltpu.SemaphoreType.DMA,
            pltpu.SemaphoreType.DMA,
        ],
        compiler_params=pltpu.CompilerParams(use_tc_tiling_on_sc=False),
    )
    def k(tab_hbm, idx_hbm, out_hbm, idx_v, rows0, rows1, sem0, sem1):
        wid = lax.axis_index("s") * 2 + lax.axis_index("c")
        base = wid * CHUNK
        pltpu.sync_copy(idx_hbm.at[:, pl.ds(base, CHUNK)], idx_v)

        def step(t, _):
            u0 = 2 * t
            u1 = 2 * t + 1
            g0 = pltpu.async_copy(tab_hbm.at[u0].at[idx_v.at[u0]], rows0, sem0)
            g1 = pltpu.async_copy(tab_hbm.at[u1].at[idx_v.at[u1]], rows1, sem1)
            g0.wait()
            pltpu.sync_copy(rows0, out_hbm.at[u0, pl.ds(base, CHUNK), pl.ds(0, D)])
            g1.wait()
            pltpu.sync_copy(rows1, out_hbm.at[u1, pl.ds(base, CHUNK), pl.ds(0, D)])
            return _
        lax.fori_loop(0, NF // 2, step, 0)

    return k(tab, idx)


def _dense_body(dx_ref, ly_ref, bw0, bb0, bw1, bb1, bw2, bb2,
                w0a, *rest):
    wbs = rest[:NF]
    tb0, tw1, tb1, tw2, tb2, out_ref = rest[NF:]

    f32 = jnp.float32
    x = dx_ref[...]
    x = jnp.maximum(jnp.dot(x, bw0[...], preferred_element_type=f32) + bb0[...], 0.0)
    x = jnp.maximum(jnp.dot(x, bw1[...], preferred_element_type=f32) + bb1[...], 0.0)
    x = jnp.maximum(jnp.dot(x, bw2[...], preferred_element_type=f32) + bb2[...], 0.0)

    ly = ly_ref[:, :, 0:D]
    t3 = jnp.concatenate([x[None], ly], axis=0).astype(jnp.bfloat16)

    acc = lax.dot_general(w0a[...], x, (((1,), (1,)), ((), ())),
                          preferred_element_type=f32)
    acc = acc + tb0[...]
    for i in range(1, 27):
        p = jnp.sum(t3[i] * t3[:i], axis=-1)
        acc = acc + lax.dot_general(wbs[i - 1][...], p,
                                    (((1,), (0,)), ((), ())),
                                    preferred_element_type=f32)
    h = jnp.maximum(acc, 0.0)
    h = jnp.maximum(jnp.dot(tw1[...], h, preferred_element_type=f32) + tb1[...], 0.0)
    h = jnp.dot(tw2[...], h, preferred_element_type=f32) + tb2[...]
    out_ref[...] = 1.0 / (1.0 + jnp.exp(-h))


def _dense(dense_x, ly, bw0t, bb0, bw1t, bb1, bw2t, bb2,
           w0a, wbs, tb0, tw1, tb1, tw2, tb2):
    full = lambda s: pl.BlockSpec(s, lambda i: (0,) * len(s))
    in_specs = [
        pl.BlockSpec((BB, 13), lambda i: (i, 0)),
        pl.BlockSpec((NF, BB, 2 * D), lambda i: (0, i, 0)),
        full((13, 512)), full((1, 512)),
        full((512, 256)), full((1, 256)),
        full((256, 64)), full((1, 64)),
        full((512, 64)),
    ]
    in_specs += [full((512, i)) for i in range(1, 27)]
    in_specs += [full((512, 1)), full((256, 512)), full((256, 1)),
                 full((1, 256)), full((1, 1))]
    out = pl.pallas_call(
        _dense_body,
        grid=(GRID,),
        in_specs=in_specs,
        out_specs=pl.BlockSpec((1, BB), lambda i: (0, i)),
        out_shape=jax.ShapeDtypeStruct((1, B), jnp.float32),
        compiler_params=pltpu.CompilerParams(
            dimension_semantics=("arbitrary",)),
    )(dense_x, ly, bw0t, bb0, bw1t, bb1, bw2t, bb2, w0a, *wbs,
      tb0, tw1, tb1, tw2, tb2)
    return out


def kernel(dense_x, lS_i, emb_tables,
           bot_w0, bot_b0, bot_w1, bot_b1, bot_w2, bot_b2,
           top_w0, top_b0, top_w1, top_b1, top_w2, top_b2):
    idx = lS_i.astype(jnp.int32)
    ly = _sc_gather(emb_tables, idx)

    bw0t, bw1t, bw2t = bot_w0.T, bot_w1.T, bot_w2.T
    bb0, bb1, bb2 = (bot_b0.reshape(1, -1), bot_b1.reshape(1, -1),
                     bot_b2.reshape(1, -1))
    w0a = top_w0[:, :D]
    offs = [i * (i - 1) // 2 for i in range(27)]
    wbs = [top_w0[:, D + offs[i]: D + offs[i] + i].astype(jnp.bfloat16)
           for i in range(1, 27)]
    tb0 = top_b0.reshape(-1, 1)
    tb1 = top_b1.reshape(-1, 1)
    tb2 = top_b2.reshape(-1, 1)

    out = _dense(dense_x, ly, bw0t, bb0, bw1t, bb1, bw2t, bb2,
                 w0a, wbs, tb0, top_w1, tb1, top_w2, tb2)
    return out.reshape(B, 1)
